# baseline (speedup 1.0000x reference)
import functools

import jax
import jax.numpy as jnp
from jax import lax
from jax.experimental import pallas as pl
from jax.experimental.pallas import tpu as pltpu

NUM_CODES = 8192
CODE_DIM = 256
BETA = 0.1
N_ROWS = 16 * 1024

TR = 256
TC_CHUNK = 1024
N_CHUNKS = NUM_CODES // TC_CHUNK
GRID_R = N_ROWS // TR


def _argmin_kernel(z_ref, w_ref, idx_ref, loss_ref, acc_ref):
    r = pl.program_id(0)
    z = z_ref[...]
    d0 = jnp.sum(z * z, axis=1, keepdims=True)

    minval = jnp.full((TR, 1), jnp.inf, jnp.float32)
    minidx = jnp.zeros((TR, 1), jnp.int32)
    for j in range(N_CHUNKS):
        wc = w_ref[j * TC_CHUNK:(j + 1) * TC_CHUNK, :]
        d1 = jnp.sum(wc * wc, axis=1, keepdims=True).reshape(1, TC_CHUNK)
        zw = lax.dot_general(z, wc, (((1,), (1,)), ((), ())),
                             preferred_element_type=jnp.float32)
        d = (d0 + d1) - 2.0 * zw
        m = jnp.min(d, axis=1, keepdims=True)
        ii = lax.broadcasted_iota(jnp.int32, d.shape, 1) + (j * TC_CHUNK)
        li = jnp.min(jnp.where(d == m, ii, jnp.int32(2**30)),
                     axis=1, keepdims=True)
        take = m < minval
        minidx = jnp.where(take, li, minidx)
        minval = jnp.where(take, m, minval)

    idx_ref[...] = minidx

    partial = jnp.sum(minval, axis=(0, 1), keepdims=True)

    @pl.when(r == 0)
    def _():
        acc_ref[...] = jnp.zeros((1, 1), jnp.float32)

    acc_ref[...] += partial
    loss_ref[...] = acc_ref[...] * ((1.0 + BETA) / (N_ROWS * CODE_DIM))


def _distance_argmin(z_r, W):
    return pl.pallas_call(
        _argmin_kernel,
        grid=(GRID_R,),
        in_specs=[
            pl.BlockSpec((TR, CODE_DIM), lambda r: (r, 0)),
            pl.BlockSpec((NUM_CODES, CODE_DIM), lambda r: (0, 0)),
        ],
        out_specs=[
            pl.BlockSpec((TR, 1), lambda r: (r, 0)),
            pl.BlockSpec((1, 1), lambda r: (0, 0)),
        ],
        out_shape=[
            jax.ShapeDtypeStruct((N_ROWS, 1), jnp.int32),
            jax.ShapeDtypeStruct((1, 1), jnp.float32),
        ],
        scratch_shapes=[pltpu.VMEM((1, 1), jnp.float32)],
    )(z_r, W)


def _make_sc_gather():
    from jax.experimental.pallas import tpu_sc as plsc

    info = plsc.get_sparse_core_info()
    NC, NS = info.num_cores, info.num_subcores
    NW = NC * NS
    b_per_w = N_ROWS // NW
    CH = 256
    n_chunks = b_per_w // CH
    mesh = plsc.VectorSubcoreMesh(core_axis_name="c", subcore_axis_name="s")

    @functools.partial(
        pl.kernel,
        mesh=mesh,
        out_type=jax.ShapeDtypeStruct((N_ROWS, CODE_DIM), jnp.float32),
        scratch_types=[
            pltpu.VMEM((CH,), jnp.int32),
            pltpu.VMEM((CH, CODE_DIM), jnp.float32),
            pltpu.SemaphoreType.DMA,
        ],
    )
    def gather_k(idx_hbm, w_hbm, out_hbm, idx_v, rows_v, sem):
        wid = lax.axis_index("s") * NC + lax.axis_index("c")
        for chunk in range(n_chunks):
            base = wid * b_per_w + chunk * CH
            pltpu.sync_copy(idx_hbm.at[pl.ds(base, CH)], idx_v)
            pltpu.async_copy(w_hbm.at[idx_v], rows_v, sem).wait()
            pltpu.sync_copy(rows_v, out_hbm.at[pl.ds(base, CH)])

    return gather_k


_sc_gather = _make_sc_gather()


def kernel(z, W):
    z_r = z.reshape(-1, CODE_DIM)
    idx, loss = _distance_argmin(z_r, W)
    z_q = _sc_gather(idx[:, 0], W).reshape(z.shape)
    z_q_out = z + lax.stop_gradient(z_q - z)
    return (z_q_out, jnp.reshape(loss, ()))


# speedup vs baseline: 1.3598x; 1.3598x over previous
import functools

import jax
import jax.numpy as jnp
from jax import lax
from jax.experimental import pallas as pl
from jax.experimental.pallas import tpu as pltpu

NUM_CODES = 8192
CODE_DIM = 256
BETA = 0.1
N_ROWS = 16 * 1024

TR = 256
TC_CHUNK = 1024
N_CHUNKS = NUM_CODES // TC_CHUNK
GRID_R = N_ROWS // TR


def _argmin_kernel(z_ref, w2_ref, idx_ref, loss_ref, acc_ref, d1_ref):
    r = pl.program_id(0)
    z = z_ref[...]
    d0 = jnp.sum(z * z, axis=1, keepdims=True)

    @pl.when(r == 0)
    def _():
        for j in range(N_CHUNKS):
            wc2 = w2_ref[j * TC_CHUNK:(j + 1) * TC_CHUNK, :]
            d1c = jnp.sum(wc2 * wc2, axis=1, keepdims=True).reshape(1, TC_CHUNK)
            d1_ref[:, j * TC_CHUNK:(j + 1) * TC_CHUNK] = d1c * 0.25

    minval = jnp.full((TR, 1), jnp.inf, jnp.float32)
    minidx = jnp.zeros((TR, 1), jnp.int32)
    for j in range(N_CHUNKS):
        wc2 = w2_ref[j * TC_CHUNK:(j + 1) * TC_CHUNK, :]
        d1 = d1_ref[:, j * TC_CHUNK:(j + 1) * TC_CHUNK]
        zw2 = lax.dot_general(z, wc2, (((1,), (1,)), ((), ())),
                              preferred_element_type=jnp.float32)
        d = (d0 + d1) - zw2
        m = jnp.min(d, axis=1, keepdims=True)
        ii = lax.broadcasted_iota(jnp.int32, d.shape, 1) + (j * TC_CHUNK)
        li = jnp.min(jnp.where(d == m, ii, jnp.int32(2**30)),
                     axis=1, keepdims=True)
        take = m < minval
        minidx = jnp.where(take, li, minidx)
        minval = jnp.where(take, m, minval)

    idx_ref[...] = minidx

    partial = jnp.sum(minval, axis=(0, 1), keepdims=True)

    @pl.when(r == 0)
    def _():
        acc_ref[...] = jnp.zeros((1, 1), jnp.float32)

    acc_ref[...] += partial
    loss_ref[...] = acc_ref[...] * ((1.0 + BETA) / (N_ROWS * CODE_DIM))


def _distance_argmin(z_r, W2):
    return pl.pallas_call(
        _argmin_kernel,
        grid=(GRID_R,),
        in_specs=[
            pl.BlockSpec((TR, CODE_DIM), lambda r: (r, 0)),
            pl.BlockSpec((NUM_CODES, CODE_DIM), lambda r: (0, 0)),
        ],
        out_specs=[
            pl.BlockSpec((TR, 1), lambda r: (r, 0)),
            pl.BlockSpec((1, 1), lambda r: (0, 0)),
        ],
        out_shape=[
            jax.ShapeDtypeStruct((N_ROWS, 1), jnp.int32),
            jax.ShapeDtypeStruct((1, 1), jnp.float32),
        ],
        scratch_shapes=[pltpu.VMEM((1, 1), jnp.float32),
                        pltpu.VMEM((1, NUM_CODES), jnp.float32)],
    )(z_r, W2)


def _make_sc_gather():
    from jax.experimental.pallas import tpu_sc as plsc

    info = plsc.get_sparse_core_info()
    NC, NS = info.num_cores, info.num_subcores
    NW = NC * NS
    b_per_w = N_ROWS // NW
    CH = 256
    n_chunks = b_per_w // CH
    mesh = plsc.VectorSubcoreMesh(core_axis_name="c", subcore_axis_name="s")

    @functools.partial(
        pl.kernel,
        mesh=mesh,
        out_type=jax.ShapeDtypeStruct((N_ROWS, CODE_DIM), jnp.float32),
        scratch_types=[
            pltpu.VMEM((CH,), jnp.int32),
            pltpu.VMEM((CH, CODE_DIM), jnp.float32),
            pltpu.SemaphoreType.DMA,
        ],
    )
    def gather_k(idx_hbm, w_hbm, out_hbm, idx_v, rows_v, sem):
        wid = lax.axis_index("s") * NC + lax.axis_index("c")
        for chunk in range(n_chunks):
            base = wid * b_per_w + chunk * CH
            pltpu.sync_copy(idx_hbm.at[pl.ds(base, CH)], idx_v)
            pltpu.async_copy(w_hbm.at[idx_v], rows_v, sem).wait()
            pltpu.sync_copy(rows_v, out_hbm.at[pl.ds(base, CH)])

    return gather_k


_sc_gather = _make_sc_gather()


def kernel(z, W):
    z_r = z.reshape(-1, CODE_DIM)
    idx, loss = _distance_argmin(z_r, W + W)
    z_q = _sc_gather(idx[:, 0], W).reshape(z.shape)
    z_q_out = z + lax.stop_gradient(z_q - z)
    return (z_q_out, jnp.reshape(loss, ()))


# speedup vs baseline: 1.5084x; 1.1093x over previous
import functools

import jax
import jax.numpy as jnp
from jax import lax
from jax.experimental import pallas as pl
from jax.experimental.pallas import tpu as pltpu

NUM_CODES = 8192
CODE_DIM = 256
BETA = 0.1
N_ROWS = 16 * 1024

TR = 256
TC_CHUNK = 1024
N_CHUNKS = NUM_CODES // TC_CHUNK
GRID_R = N_ROWS // TR


def _argmin_kernel(z_ref, w2_ref, idx_ref, loss_ref, acc_ref, d1_ref):
    r = pl.program_id(0)
    z = z_ref[...]
    d0 = jnp.sum(z * z, axis=1, keepdims=True)

    @pl.when(r == 0)
    def _():
        for j in range(N_CHUNKS):
            wc2 = w2_ref[j * TC_CHUNK:(j + 1) * TC_CHUNK, :]
            d1c = jnp.sum(wc2 * wc2, axis=1, keepdims=True).reshape(1, TC_CHUNK)
            d1_ref[:, j * TC_CHUNK:(j + 1) * TC_CHUNK] = d1c * 0.25

    g = jnp.full((TR, TC_CHUNK), jnp.inf, jnp.float32)
    c = jnp.zeros((TR, TC_CHUNK), jnp.int32)
    for j in range(N_CHUNKS):
        wc2 = w2_ref[j * TC_CHUNK:(j + 1) * TC_CHUNK, :]
        d1 = d1_ref[:, j * TC_CHUNK:(j + 1) * TC_CHUNK]
        zw2 = lax.dot_general(z, wc2, (((1,), (1,)), ((), ())),
                              preferred_element_type=jnp.float32)
        d = (d0 + d1) - zw2
        upd = d < g
        g = jnp.where(upd, d, g)
        c = jnp.where(upd, jnp.int32(j), c)

    m = jnp.min(g, axis=1, keepdims=True)
    lane = lax.broadcasted_iota(jnp.int32, g.shape, 1)
    key = jnp.where(g == m, c * TC_CHUNK + lane, jnp.int32(2**30))
    minidx = jnp.min(key, axis=1, keepdims=True)
    idx_ref[...] = minidx

    partial = jnp.sum(m, axis=(0, 1), keepdims=True)

    @pl.when(r == 0)
    def _():
        acc_ref[...] = jnp.zeros((1, 1), jnp.float32)

    acc_ref[...] += partial
    loss_ref[...] = acc_ref[...] * ((1.0 + BETA) / (N_ROWS * CODE_DIM))


def _distance_argmin(z_r, W2):
    return pl.pallas_call(
        _argmin_kernel,
        grid=(GRID_R,),
        in_specs=[
            pl.BlockSpec((TR, CODE_DIM), lambda r: (r, 0)),
            pl.BlockSpec((NUM_CODES, CODE_DIM), lambda r: (0, 0)),
        ],
        out_specs=[
            pl.BlockSpec((TR, 1), lambda r: (r, 0)),
            pl.BlockSpec((1, 1), lambda r: (0, 0)),
        ],
        out_shape=[
            jax.ShapeDtypeStruct((N_ROWS, 1), jnp.int32),
            jax.ShapeDtypeStruct((1, 1), jnp.float32),
        ],
        scratch_shapes=[pltpu.VMEM((1, 1), jnp.float32),
                        pltpu.VMEM((1, NUM_CODES), jnp.float32)],
    )(z_r, W2)


def _make_sc_gather():
    from jax.experimental.pallas import tpu_sc as plsc

    info = plsc.get_sparse_core_info()
    NC, NS = info.num_cores, info.num_subcores
    NW = NC * NS
    b_per_w = N_ROWS // NW
    CH = 256
    n_chunks = b_per_w // CH
    mesh = plsc.VectorSubcoreMesh(core_axis_name="c", subcore_axis_name="s")

    @functools.partial(
        pl.kernel,
        mesh=mesh,
        out_type=jax.ShapeDtypeStruct((N_ROWS, CODE_DIM), jnp.float32),
        scratch_types=[
            pltpu.VMEM((CH,), jnp.int32),
            pltpu.VMEM((CH, CODE_DIM), jnp.float32),
            pltpu.SemaphoreType.DMA,
        ],
    )
    def gather_k(idx_hbm, w_hbm, out_hbm, idx_v, rows_v, sem):
        wid = lax.axis_index("s") * NC + lax.axis_index("c")
        for chunk in range(n_chunks):
            base = wid * b_per_w + chunk * CH
            pltpu.sync_copy(idx_hbm.at[pl.ds(base, CH)], idx_v)
            pltpu.async_copy(w_hbm.at[idx_v], rows_v, sem).wait()
            pltpu.sync_copy(rows_v, out_hbm.at[pl.ds(base, CH)])

    return gather_k


_sc_gather = _make_sc_gather()


def kernel(z, W):
    z_r = z.reshape(-1, CODE_DIM)
    idx, loss = _distance_argmin(z_r, W + W)
    z_q = _sc_gather(idx[:, 0], W).reshape(z.shape)
    z_q_out = z + lax.stop_gradient(z_q - z)
    return (z_q_out, jnp.reshape(loss, ()))


# speedup vs baseline: 1.6251x; 1.0774x over previous
import functools

import jax
import jax.numpy as jnp
from jax import lax
from jax.experimental import pallas as pl
from jax.experimental.pallas import tpu as pltpu

NUM_CODES = 8192
CODE_DIM = 256
BETA = 0.1
N_ROWS = 16 * 1024

TR = 512
TC_CHUNK = 1024
N_CHUNKS = NUM_CODES // TC_CHUNK
GRID_R = N_ROWS // TR


def _argmin_kernel(z_ref, w2_ref, idx_ref, loss_ref, acc_ref, d1_ref):
    r = pl.program_id(0)
    z = z_ref[...]
    d0 = jnp.sum(z * z, axis=1, keepdims=True)

    @pl.when(r == 0)
    def _():
        for j in range(N_CHUNKS):
            wc2 = w2_ref[j * TC_CHUNK:(j + 1) * TC_CHUNK, :]
            d1c = jnp.sum(wc2 * wc2, axis=1, keepdims=True).reshape(1, TC_CHUNK)
            d1_ref[:, j * TC_CHUNK:(j + 1) * TC_CHUNK] = d1c * 0.25

    g = jnp.full((TR, TC_CHUNK), jnp.inf, jnp.float32)
    c = jnp.zeros((TR, TC_CHUNK), jnp.int32)
    for j in range(N_CHUNKS):
        wc2 = w2_ref[j * TC_CHUNK:(j + 1) * TC_CHUNK, :]
        d1 = d1_ref[:, j * TC_CHUNK:(j + 1) * TC_CHUNK]
        zw2 = lax.dot_general(z, wc2, (((1,), (1,)), ((), ())),
                              preferred_element_type=jnp.float32)
        d = (d0 + d1) - zw2
        upd = d < g
        g = jnp.where(upd, d, g)
        c = jnp.where(upd, jnp.int32(j), c)

    m = jnp.min(g, axis=1, keepdims=True)
    lane = lax.broadcasted_iota(jnp.int32, g.shape, 1)
    key = jnp.where(g == m, c * TC_CHUNK + lane, jnp.int32(2**30))
    minidx = jnp.min(key, axis=1, keepdims=True)
    idx_ref[...] = minidx

    partial = jnp.sum(m, axis=(0, 1), keepdims=True)

    @pl.when(r == 0)
    def _():
        acc_ref[...] = jnp.zeros((1, 1), jnp.float32)

    acc_ref[...] += partial
    loss_ref[...] = acc_ref[...] * ((1.0 + BETA) / (N_ROWS * CODE_DIM))


def _distance_argmin(z_r, W2):
    return pl.pallas_call(
        _argmin_kernel,
        grid=(GRID_R,),
        in_specs=[
            pl.BlockSpec((TR, CODE_DIM), lambda r: (r, 0)),
            pl.BlockSpec((NUM_CODES, CODE_DIM), lambda r: (0, 0)),
        ],
        out_specs=[
            pl.BlockSpec((TR, 1), lambda r: (r, 0)),
            pl.BlockSpec((1, 1), lambda r: (0, 0)),
        ],
        out_shape=[
            jax.ShapeDtypeStruct((N_ROWS, 1), jnp.int32),
            jax.ShapeDtypeStruct((1, 1), jnp.float32),
        ],
        scratch_shapes=[pltpu.VMEM((1, 1), jnp.float32),
                        pltpu.VMEM((1, NUM_CODES), jnp.float32)],
    )(z_r, W2)


def _make_sc_gather():
    from jax.experimental.pallas import tpu_sc as plsc

    info = plsc.get_sparse_core_info()
    NC, NS = info.num_cores, info.num_subcores
    NW = NC * NS
    b_per_w = N_ROWS // NW
    CH = 256
    n_chunks = b_per_w // CH
    mesh = plsc.VectorSubcoreMesh(core_axis_name="c", subcore_axis_name="s")

    @functools.partial(
        pl.kernel,
        mesh=mesh,
        out_type=jax.ShapeDtypeStruct((N_ROWS, CODE_DIM), jnp.float32),
        scratch_types=[
            pltpu.VMEM((CH,), jnp.int32),
            pltpu.VMEM((CH, CODE_DIM), jnp.float32),
            pltpu.SemaphoreType.DMA,
        ],
    )
    def gather_k(idx_hbm, w_hbm, out_hbm, idx_v, rows_v, sem):
        wid = lax.axis_index("s") * NC + lax.axis_index("c")
        for chunk in range(n_chunks):
            base = wid * b_per_w + chunk * CH
            pltpu.sync_copy(idx_hbm.at[pl.ds(base, CH)], idx_v)
            pltpu.async_copy(w_hbm.at[idx_v], rows_v, sem).wait()
            pltpu.sync_copy(rows_v, out_hbm.at[pl.ds(base, CH)])

    return gather_k


_sc_gather = _make_sc_gather()


def kernel(z, W):
    z_r = z.reshape(-1, CODE_DIM)
    idx, loss = _distance_argmin(z_r, W + W)
    z_q = _sc_gather(idx[:, 0], W).reshape(z.shape)
    z_q_out = z + lax.stop_gradient(z_q - z)
    return (z_q_out, jnp.reshape(loss, ()))


# speedup vs baseline: 1.6478x; 1.0140x over previous
import functools

import jax
import jax.numpy as jnp
from jax import lax
from jax.experimental import pallas as pl
from jax.experimental.pallas import tpu as pltpu

NUM_CODES = 8192
CODE_DIM = 256
BETA = 0.1
N_ROWS = 16 * 1024

TR = 1024
TC_CHUNK = 1024
N_CHUNKS = NUM_CODES // TC_CHUNK
GRID_R = N_ROWS // TR


def _argmin_kernel(z_ref, w2_ref, idx_ref, loss_ref, acc_ref, d1_ref):
    r = pl.program_id(0)
    z = z_ref[...]
    d0 = jnp.sum(z * z, axis=1, keepdims=True)

    @pl.when(r == 0)
    def _():
        for j in range(N_CHUNKS):
            wc2 = w2_ref[j * TC_CHUNK:(j + 1) * TC_CHUNK, :]
            d1c = jnp.sum(wc2 * wc2, axis=1, keepdims=True).reshape(1, TC_CHUNK)
            d1_ref[:, j * TC_CHUNK:(j + 1) * TC_CHUNK] = d1c * 0.25

    g = jnp.full((TR, TC_CHUNK), jnp.inf, jnp.float32)
    c = jnp.zeros((TR, TC_CHUNK), jnp.int32)
    for j in range(N_CHUNKS):
        wc2 = w2_ref[j * TC_CHUNK:(j + 1) * TC_CHUNK, :]
        d1 = d1_ref[:, j * TC_CHUNK:(j + 1) * TC_CHUNK]
        zw2 = lax.dot_general(z, wc2, (((1,), (1,)), ((), ())),
                              preferred_element_type=jnp.float32)
        d = (d0 + d1) - zw2
        upd = d < g
        g = jnp.where(upd, d, g)
        c = jnp.where(upd, jnp.int32(j), c)

    m = jnp.min(g, axis=1, keepdims=True)
    lane = lax.broadcasted_iota(jnp.int32, g.shape, 1)
    key = jnp.where(g == m, c * TC_CHUNK + lane, jnp.int32(2**30))
    minidx = jnp.min(key, axis=1, keepdims=True)
    idx_ref[...] = minidx

    partial = jnp.sum(m, axis=(0, 1), keepdims=True)

    @pl.when(r == 0)
    def _():
        acc_ref[...] = jnp.zeros((1, 1), jnp.float32)

    acc_ref[...] += partial
    loss_ref[...] = acc_ref[...] * ((1.0 + BETA) / (N_ROWS * CODE_DIM))


def _distance_argmin(z_r, W2):
    return pl.pallas_call(
        _argmin_kernel,
        grid=(GRID_R,),
        in_specs=[
            pl.BlockSpec((TR, CODE_DIM), lambda r: (r, 0)),
            pl.BlockSpec((NUM_CODES, CODE_DIM), lambda r: (0, 0)),
        ],
        out_specs=[
            pl.BlockSpec((TR, 1), lambda r: (r, 0)),
            pl.BlockSpec((1, 1), lambda r: (0, 0)),
        ],
        out_shape=[
            jax.ShapeDtypeStruct((N_ROWS, 1), jnp.int32),
            jax.ShapeDtypeStruct((1, 1), jnp.float32),
        ],
        scratch_shapes=[pltpu.VMEM((1, 1), jnp.float32),
                        pltpu.VMEM((1, NUM_CODES), jnp.float32)],
    )(z_r, W2)


def _make_sc_gather():
    from jax.experimental.pallas import tpu_sc as plsc

    info = plsc.get_sparse_core_info()
    NC, NS = info.num_cores, info.num_subcores
    NW = NC * NS
    b_per_w = N_ROWS // NW
    CH = 256
    n_chunks = b_per_w // CH
    mesh = plsc.VectorSubcoreMesh(core_axis_name="c", subcore_axis_name="s")

    @functools.partial(
        pl.kernel,
        mesh=mesh,
        out_type=jax.ShapeDtypeStruct((N_ROWS, CODE_DIM), jnp.float32),
        scratch_types=[
            pltpu.VMEM((CH,), jnp.int32),
            pltpu.VMEM((CH, CODE_DIM), jnp.float32),
            pltpu.SemaphoreType.DMA,
        ],
    )
    def gather_k(idx_hbm, w_hbm, out_hbm, idx_v, rows_v, sem):
        wid = lax.axis_index("s") * NC + lax.axis_index("c")
        for chunk in range(n_chunks):
            base = wid * b_per_w + chunk * CH
            pltpu.sync_copy(idx_hbm.at[pl.ds(base, CH)], idx_v)
            pltpu.async_copy(w_hbm.at[idx_v], rows_v, sem).wait()
            pltpu.sync_copy(rows_v, out_hbm.at[pl.ds(base, CH)])

    return gather_k


_sc_gather = _make_sc_gather()


def kernel(z, W):
    z_r = z.reshape(-1, CODE_DIM)
    idx, loss = _distance_argmin(z_r, W + W)
    z_q = _sc_gather(idx[:, 0], W).reshape(z.shape)
    z_q_out = z + lax.stop_gradient(z_q - z)
    return (z_q_out, jnp.reshape(loss, ()))
